# Initial kernel scaffold; baseline (speedup 1.0000x reference)
import jax
import jax.numpy as jnp
from jax.experimental import pallas as pl


def kernel(x, edge_index, edge_attr, u, batch, W1, b1, W2, b2):
    raise NotImplementedError("write your pallas kernel here")


# speedup vs baseline: 15.0996x; 15.0996x over previous
import functools

import jax
import jax.numpy as jnp
from jax import lax
from jax.experimental import pallas as pl
from jax.experimental.pallas import tpu as pltpu
from jax.experimental.pallas import tpu_sc as plsc

B = 64
D = 128
N = 10000
E = 320000
NC = 2
NS = 16
NW = NC * NS

SUB = 80
CHUNK = 400
E_PER_W = E // NW
N_ECHUNKS = E_PER_W // CHUNK
N_NBLOCKS = N // SUB


def _histo_update(cnt_ref, seg_vec):
    r, m = plsc.scan_count(seg_vec)
    plsc.addupdate_scatter(cnt_ref, [seg_vec], r.astype(jnp.float32), mask=m)


def _sc_body(ei1_hbm, x_hbm, batch_hbm, eattr_hbm,
             nacc_out, ncnt_out, eacc_out, ecnt_out,
             batch_v, ei_v, seg_v, attr_v, ncnt_v, ecnt_v,
             nacc_s, eacc_s):
    c = lax.axis_index("c")
    s = lax.axis_index("s")
    w = s * NC + c

    zero16 = jnp.zeros((16,), jnp.float32)
    for t in range(B // 16):
        ncnt_v[pl.ds(t * 16, 16)] = zero16
        ecnt_v[pl.ds(t * 16, 16)] = zero16

    pltpu.sync_copy(batch_hbm, batch_v)

    @pl.when(s == 0)
    def _():
        def zrow(i, _):
            for j in range(D // 16):
                attr_v[i, pl.ds(j * 16, 16)] = zero16
            return 0
        lax.fori_loop(0, B, zrow, 0)
        pltpu.sync_copy(attr_v.at[pl.ds(0, B)], nacc_s)
        pltpu.sync_copy(attr_v.at[pl.ds(0, B)], eacc_s)

    plsc.subcore_barrier()

    for it in range((N_NBLOCKS + NW - 1) // NW):
        blk = w + it * NW

        @pl.when(blk < N_NBLOCKS)
        def _():
            base = blk * SUB
            pltpu.sync_copy(batch_hbm.at[pl.ds(base, SUB)], seg_v.at[0])
            pltpu.sync_copy(x_hbm.at[pl.ds(base, SUB)], attr_v.at[pl.ds(0, SUB)])
            pltpu.sync_copy(attr_v.at[pl.ds(0, SUB)], nacc_s.at[seg_v.at[0]],
                            add=True)
            for t in range(SUB // 16):
                _histo_update(ncnt_v, seg_v[0, pl.ds(t * 16, 16)])

    ebase = w * E_PER_W

    def echunk(k, _):
        cb = ebase + k * CHUNK
        pltpu.sync_copy(ei1_hbm.at[pl.ds(cb, CHUNK)], ei_v)
        pltpu.sync_copy(eattr_hbm.at[pl.ds(cb, CHUNK)], attr_v)
        for j in range(CHUNK // SUB):
            for t in range(SUB // 16):
                idx = ei_v[pl.ds(j * SUB + t * 16, 16)]
                seg = plsc.load_gather(batch_v, [idx])
                seg_v[j, pl.ds(t * 16, 16)] = seg
                _histo_update(ecnt_v, seg)
        for j in range(CHUNK // SUB):
            pltpu.sync_copy(attr_v.at[pl.ds(j * SUB, SUB)],
                            eacc_s.at[seg_v.at[j]], add=True)
        return 0

    lax.fori_loop(0, N_ECHUNKS, echunk, 0)

    pltpu.sync_copy(ncnt_v, ncnt_out.at[c, s])
    pltpu.sync_copy(ecnt_v, ecnt_out.at[c, s])

    plsc.subcore_barrier()

    @pl.when(s == 0)
    def _():
        pltpu.sync_copy(nacc_s, nacc_out.at[c])
        pltpu.sync_copy(eacc_s, eacc_out.at[c])


_sc_segment_sums = functools.partial(
    pl.kernel,
    out_type=(
        jax.ShapeDtypeStruct((NC, B, D), jnp.float32),
        jax.ShapeDtypeStruct((NC, NS, B), jnp.float32),
        jax.ShapeDtypeStruct((NC, B, D), jnp.float32),
        jax.ShapeDtypeStruct((NC, NS, B), jnp.float32),
    ),
    mesh=plsc.VectorSubcoreMesh(core_axis_name="c", subcore_axis_name="s"),
    compiler_params=pltpu.CompilerParams(needs_layout_passes=False),
    scratch_types=[
        pltpu.VMEM((N,), jnp.int32),
        pltpu.VMEM((CHUNK,), jnp.int32),
        pltpu.VMEM((CHUNK // SUB, SUB), jnp.int32),
        pltpu.VMEM((CHUNK, D), jnp.float32),
        pltpu.VMEM((B,), jnp.float32),
        pltpu.VMEM((B,), jnp.float32),
        pltpu.VMEM_SHARED((B, D), jnp.float32),
        pltpu.VMEM_SHARED((B, D), jnp.float32),
    ],
)(_sc_body)


def _mlp_body(nacc, ncnt, eacc, ecnt, u, w1u, w1n, w1e, b1, w2, b2, out):
    nsum = nacc[0] + nacc[1]
    esum = eacc[0] + eacc[1]
    ncount = jnp.sum(ncnt[...], axis=(0, 1)).reshape(B, 1)
    ecount = jnp.sum(ecnt[...], axis=(0, 1)).reshape(B, 1)
    nmean = nsum / jnp.maximum(ncount, 1.0)
    emean = esum / jnp.maximum(ecount, 1.0)
    h = (jnp.dot(u[...], w1u[...], preferred_element_type=jnp.float32)
         + jnp.dot(nmean, w1n[...], preferred_element_type=jnp.float32)
         + jnp.dot(emean, w1e[...], preferred_element_type=jnp.float32)
         + b1[...])
    h = jnp.maximum(h, 0.0)
    out[...] = jnp.dot(h, w2[...], preferred_element_type=jnp.float32) + b2[...]


def kernel(x, edge_index, edge_attr, u, batch, W1, b1, W2, b2):
    ei1 = edge_index[1].astype(jnp.int32)
    batch_i = batch.astype(jnp.int32)
    nacc, ncnt, eacc, ecnt = _sc_segment_sums(ei1, x, batch_i, edge_attr)

    U = u.shape[1]
    w1u = W1[:U]
    w1n = W1[U:U + D]
    w1e = W1[U + D:]
    out = pl.pallas_call(
        _mlp_body,
        out_shape=jax.ShapeDtypeStruct((B, D), jnp.float32),
    )(nacc, ncnt, eacc, ecnt, u, w1u, w1n, w1e,
      b1.reshape(1, D), W2, b2.reshape(1, D))
    return out


# speedup vs baseline: 19.6409x; 1.3008x over previous
import functools

import jax
import jax.numpy as jnp
from jax import lax
from jax.experimental import pallas as pl
from jax.experimental.pallas import tpu as pltpu
from jax.experimental.pallas import tpu_sc as plsc

B = 64
D = 128
N = 10000
E = 320000
NC = 2
NS = 16
NW = NC * NS

SUB = 80
NSUB = 5
CHUNK = SUB * NSUB
E_PER_W = E // NW
N_ECHUNKS = E_PER_W // CHUNK
N_NBLOCKS = N // SUB


def _histo_update(cnt_ref, seg_vec):
    r, m = plsc.scan_count(seg_vec)
    plsc.addupdate_scatter(cnt_ref, [seg_vec], r.astype(jnp.float32), mask=m)


def _sc_body(ei1_hbm, x_hbm, batch_hbm, eattr_hbm,
             nacc_out, ncnt_out, eacc_out, ecnt_out,
             batch_v, ei_v0, ei_v1, attr_v0, attr_v1,
             sg00, sg01, sg02, sg03, sg04,
             sg10, sg11, sg12, sg13, sg14,
             ncnt_v, ecnt_v, nacc_s, eacc_s,
             gsem0, gsem1, ssem0, ssem1):
    c = lax.axis_index("c")
    s = lax.axis_index("s")
    w = s * NC + c

    ei_v = (ei_v0, ei_v1)
    attr_v = (attr_v0, attr_v1)
    seg_v = ((sg00, sg01, sg02, sg03, sg04),
             (sg10, sg11, sg12, sg13, sg14))
    gsem = (gsem0, gsem1)
    ssem = (ssem0, ssem1)

    zero16 = jnp.zeros((16,), jnp.float32)
    for t in range(B // 16):
        ncnt_v[pl.ds(t * 16, 16)] = zero16
        ecnt_v[pl.ds(t * 16, 16)] = zero16

    pltpu.sync_copy(batch_hbm, batch_v)

    @pl.when(s == 0)
    def _():
        def zrow(i, _):
            for j in range(D // 16):
                attr_v0[i, pl.ds(j * 16, 16)] = zero16
            return 0
        lax.fori_loop(0, B, zrow, 0)
        pltpu.sync_copy(attr_v0.at[pl.ds(0, B)], nacc_s)
        pltpu.sync_copy(attr_v0.at[pl.ds(0, B)], eacc_s)

    plsc.subcore_barrier()

    for it in range((N_NBLOCKS + NW - 1) // NW):
        blk = w + it * NW

        @pl.when(blk < N_NBLOCKS)
        def _():
            base = blk * SUB
            pltpu.sync_copy(batch_hbm.at[pl.ds(base, SUB)], sg00)
            pltpu.sync_copy(x_hbm.at[pl.ds(base, SUB)],
                            attr_v0.at[pl.ds(0, SUB)])
            pltpu.sync_copy(attr_v0.at[pl.ds(0, SUB)], nacc_s.at[sg00],
                            add=True)
            for t in range(SUB // 16):
                _histo_update(ncnt_v, sg00[pl.ds(t * 16, 16)])

    ebase = w * E_PER_W

    def start_gather(k, b):
        cb = ebase + k * CHUNK
        pltpu.async_copy(ei1_hbm.at[pl.ds(cb, CHUNK)], ei_v[b], gsem[b])
        pltpu.async_copy(eattr_hbm.at[pl.ds(cb, CHUNK)], attr_v[b], gsem[b])

    def wait_gather(k, b):
        cb = ebase + k * CHUNK
        pltpu.make_async_copy(ei1_hbm.at[pl.ds(cb, CHUNK)], ei_v[b],
                              gsem[b]).wait()
        pltpu.make_async_copy(eattr_hbm.at[pl.ds(cb, CHUNK)], attr_v[b],
                              gsem[b]).wait()

    def compute_segs(b):
        for j in range(NSUB):
            for t in range(SUB // 16):
                idx = ei_v[b][pl.ds(j * SUB + t * 16, 16)]
                seg = plsc.load_gather(batch_v, [idx])
                seg_v[b][j][pl.ds(t * 16, 16)] = seg
                _histo_update(ecnt_v, seg)

    def issue_scatter(b):
        for j in range(NSUB):
            pltpu.async_copy(attr_v[b].at[pl.ds(j * SUB, SUB)],
                             eacc_s.at[seg_v[b][j]], ssem[b], add=True)

    def drain_scatter(b):
        for j in range(NSUB):
            pltpu.make_async_copy(attr_v[b].at[pl.ds(j * SUB, SUB)],
                                  eacc_s.at[seg_v[b][j]], ssem[b]).wait()

    start_gather(0, 0)
    start_gather(1, 1)

    wait_gather(0, 0)
    compute_segs(0)
    issue_scatter(0)

    NPAIR = (N_ECHUNKS - 1) // 2

    def epair(k2, _):
        ka = 2 * k2 + 1
        wait_gather(ka, 1)
        compute_segs(1)
        issue_scatter(1)
        drain_scatter(0)
        start_gather(ka + 1, 0)

        kb = ka + 1
        wait_gather(kb, 0)
        compute_segs(0)
        issue_scatter(0)
        drain_scatter(1)

        @pl.when(k2 < NPAIR - 1)
        def _():
            start_gather(kb + 1, 1)
        return 0

    lax.fori_loop(0, NPAIR, epair, 0)
    drain_scatter(0)

    pltpu.sync_copy(ncnt_v, ncnt_out.at[c, s])
    pltpu.sync_copy(ecnt_v, ecnt_out.at[c, s])

    plsc.subcore_barrier()

    @pl.when(s == 0)
    def _():
        pltpu.sync_copy(nacc_s, nacc_out.at[c])
        pltpu.sync_copy(eacc_s, eacc_out.at[c])


_sc_segment_sums = functools.partial(
    pl.kernel,
    out_type=(
        jax.ShapeDtypeStruct((NC, B, D), jnp.float32),
        jax.ShapeDtypeStruct((NC, NS, B), jnp.float32),
        jax.ShapeDtypeStruct((NC, B, D), jnp.float32),
        jax.ShapeDtypeStruct((NC, NS, B), jnp.float32),
    ),
    mesh=plsc.VectorSubcoreMesh(core_axis_name="c", subcore_axis_name="s"),
    compiler_params=pltpu.CompilerParams(needs_layout_passes=False),
    scratch_types=[
        pltpu.VMEM((N,), jnp.int32),
        pltpu.VMEM((CHUNK,), jnp.int32),
        pltpu.VMEM((CHUNK,), jnp.int32),
        pltpu.VMEM((CHUNK, D), jnp.float32),
        pltpu.VMEM((CHUNK, D), jnp.float32),
    ] + [pltpu.VMEM((SUB,), jnp.int32)] * (2 * NSUB)
    + [
        pltpu.VMEM((B,), jnp.float32),
        pltpu.VMEM((B,), jnp.float32),
        pltpu.VMEM_SHARED((B, D), jnp.float32),
        pltpu.VMEM_SHARED((B, D), jnp.float32),
        pltpu.SemaphoreType.DMA,
        pltpu.SemaphoreType.DMA,
        pltpu.SemaphoreType.DMA,
        pltpu.SemaphoreType.DMA,
    ],
)(_sc_body)


def _mlp_body(nacc, ncnt, eacc, ecnt, u, w1u, w1n, w1e, b1, w2, b2, out):
    nsum = nacc[0] + nacc[1]
    esum = eacc[0] + eacc[1]
    ncount = jnp.sum(ncnt[...], axis=(0, 1)).reshape(B, 1)
    ecount = jnp.sum(ecnt[...], axis=(0, 1)).reshape(B, 1)
    nmean = nsum / jnp.maximum(ncount, 1.0)
    emean = esum / jnp.maximum(ecount, 1.0)
    h = (jnp.dot(u[...], w1u[...], preferred_element_type=jnp.float32)
         + jnp.dot(nmean, w1n[...], preferred_element_type=jnp.float32)
         + jnp.dot(emean, w1e[...], preferred_element_type=jnp.float32)
         + b1[...])
    h = jnp.maximum(h, 0.0)
    out[...] = jnp.dot(h, w2[...], preferred_element_type=jnp.float32) + b2[...]


def kernel(x, edge_index, edge_attr, u, batch, W1, b1, W2, b2):
    ei1 = edge_index[1].astype(jnp.int32)
    batch_i = batch.astype(jnp.int32)
    nacc, ncnt, eacc, ecnt = _sc_segment_sums(ei1, x, batch_i, edge_attr)

    U = u.shape[1]
    w1u = W1[:U]
    w1n = W1[U:U + D]
    w1e = W1[U + D:]
    out = pl.pallas_call(
        _mlp_body,
        out_shape=jax.ShapeDtypeStruct((B, D), jnp.float32),
    )(nacc, ncnt, eacc, ecnt, u, w1u, w1n, w1e,
      b1.reshape(1, D), W2, b2.reshape(1, D))
    return out


# speedup vs baseline: 21.5387x; 1.0966x over previous
import functools

import jax
import jax.numpy as jnp
from jax import lax
from jax.experimental import pallas as pl
from jax.experimental.pallas import tpu as pltpu
from jax.experimental.pallas import tpu_sc as plsc

B = 64
D = 128
N = 10000
E = 320000
NC = 2
NS = 16
NW = NC * NS

SUB = 80
NSUB = 5
CHUNK = SUB * NSUB
E_PER_W = E // NW
N_ECHUNKS = E_PER_W // CHUNK
N_NBLOCKS = N // SUB


def _histo_update(cnt_ref, seg_vec):
    r, m = plsc.scan_count(seg_vec)
    plsc.addupdate_scatter(cnt_ref, [seg_vec], r.astype(jnp.float32), mask=m)


def _sc_body(ei_hbm, x_hbm, batch_hbm, eattr_hbm,
             nacc_out, ncnt_out, eacc_out, ecnt_out,
             batch_v, ei_v0, ei_v1, attr_v0, attr_v1, xstage_v,
             sg00, sg01, sg02, sg03, sg04,
             sg10, sg11, sg12, sg13, sg14,
             ncnt_v, ecnt_v, nacc_s, eacc_s,
             gsem0, gsem1, ssem0, ssem1):
    c = lax.axis_index("c")
    s = lax.axis_index("s")
    w = s * NC + c

    ei_v = (ei_v0, ei_v1)
    attr_v = (attr_v0, attr_v1)
    seg_v = ((sg00, sg01, sg02, sg03, sg04),
             (sg10, sg11, sg12, sg13, sg14))
    gsem = (gsem0, gsem1)
    ssem = (ssem0, ssem1)
    ebase = w * E_PER_W
    RPT = B // NS

    def start_gather(k, b):
        cb = ebase + k * CHUNK
        pltpu.async_copy(ei_hbm.at[pl.ds(E + cb, CHUNK)], ei_v[b], gsem[b])
        pltpu.async_copy(eattr_hbm.at[pl.ds(cb, CHUNK)], attr_v[b], gsem[b])

    def wait_gather(k, b):
        cb = ebase + k * CHUNK
        pltpu.make_async_copy(ei_hbm.at[pl.ds(E + cb, CHUNK)], ei_v[b],
                              gsem[b]).wait()
        pltpu.make_async_copy(eattr_hbm.at[pl.ds(cb, CHUNK)], attr_v[b],
                              gsem[b]).wait()

    start_gather(0, 0)
    start_gather(1, 1)

    zero16 = jnp.zeros((16,), jnp.float32)
    for t in range(B // 16):
        ncnt_v[pl.ds(t * 16, 16)] = zero16
        ecnt_v[pl.ds(t * 16, 16)] = zero16
    for i in range(RPT):
        for j in range(D // 16):
            xstage_v[i, pl.ds(j * 16, 16)] = zero16

    pltpu.sync_copy(batch_hbm, batch_v)

    pltpu.sync_copy(xstage_v.at[pl.ds(0, RPT)], nacc_s.at[pl.ds(s * RPT, RPT)])
    pltpu.sync_copy(xstage_v.at[pl.ds(0, RPT)], eacc_s.at[pl.ds(s * RPT, RPT)])

    plsc.subcore_barrier()

    for it in range((N_NBLOCKS + NW - 1) // NW):
        blk = w + it * NW

        @pl.when(blk < N_NBLOCKS)
        def _():
            base = blk * SUB
            pltpu.sync_copy(batch_hbm.at[pl.ds(base, SUB)], sg00)
            pltpu.sync_copy(x_hbm.at[pl.ds(base, SUB)],
                            xstage_v.at[pl.ds(0, SUB)])
            pltpu.sync_copy(xstage_v.at[pl.ds(0, SUB)], nacc_s.at[sg00],
                            add=True)
            for t in range(SUB // 16):
                _histo_update(ncnt_v, sg00[pl.ds(t * 16, 16)])


    def compute_segs(b):
        for j in range(NSUB):
            for t in range(SUB // 16):
                idx = ei_v[b][pl.ds(j * SUB + t * 16, 16)]
                seg = plsc.load_gather(batch_v, [idx])
                seg_v[b][j][pl.ds(t * 16, 16)] = seg
                _histo_update(ecnt_v, seg)

    def issue_scatter(b):
        for j in range(NSUB):
            pltpu.async_copy(attr_v[b].at[pl.ds(j * SUB, SUB)],
                             eacc_s.at[seg_v[b][j]], ssem[b], add=True)

    def drain_scatter(b):
        for j in range(NSUB):
            pltpu.make_async_copy(attr_v[b].at[pl.ds(j * SUB, SUB)],
                                  eacc_s.at[seg_v[b][j]], ssem[b]).wait()

    wait_gather(0, 0)
    compute_segs(0)
    issue_scatter(0)

    NPAIR = (N_ECHUNKS - 1) // 2

    def epair(k2, _):
        ka = 2 * k2 + 1
        wait_gather(ka, 1)
        compute_segs(1)
        issue_scatter(1)
        drain_scatter(0)
        start_gather(ka + 1, 0)

        kb = ka + 1
        wait_gather(kb, 0)
        compute_segs(0)
        issue_scatter(0)
        drain_scatter(1)

        @pl.when(k2 < NPAIR - 1)
        def _():
            start_gather(kb + 1, 1)
        return 0

    lax.fori_loop(0, NPAIR, epair, 0)
    drain_scatter(0)

    pltpu.sync_copy(ncnt_v, ncnt_out.at[c, s])
    pltpu.sync_copy(ecnt_v, ecnt_out.at[c, s])

    plsc.subcore_barrier()

    @pl.when(s == 0)
    def _():
        pltpu.sync_copy(nacc_s, nacc_out.at[c])
        pltpu.sync_copy(eacc_s, eacc_out.at[c])


_sc_segment_sums = functools.partial(
    pl.kernel,
    out_type=(
        jax.ShapeDtypeStruct((NC, B, D), jnp.float32),
        jax.ShapeDtypeStruct((NC, NS, B), jnp.float32),
        jax.ShapeDtypeStruct((NC, B, D), jnp.float32),
        jax.ShapeDtypeStruct((NC, NS, B), jnp.float32),
    ),
    mesh=plsc.VectorSubcoreMesh(core_axis_name="c", subcore_axis_name="s"),
    compiler_params=pltpu.CompilerParams(needs_layout_passes=False),
    scratch_types=[
        pltpu.VMEM((N,), jnp.int32),
        pltpu.VMEM((CHUNK,), jnp.int32),
        pltpu.VMEM((CHUNK,), jnp.int32),
        pltpu.VMEM((CHUNK, D), jnp.float32),
        pltpu.VMEM((CHUNK, D), jnp.float32),
        pltpu.VMEM((SUB, D), jnp.float32),
    ] + [pltpu.VMEM((SUB,), jnp.int32)] * (2 * NSUB)
    + [
        pltpu.VMEM((B,), jnp.float32),
        pltpu.VMEM((B,), jnp.float32),
        pltpu.VMEM_SHARED((B, D), jnp.float32),
        pltpu.VMEM_SHARED((B, D), jnp.float32),
        pltpu.SemaphoreType.DMA,
        pltpu.SemaphoreType.DMA,
        pltpu.SemaphoreType.DMA,
        pltpu.SemaphoreType.DMA,
    ],
)(_sc_body)


def _mlp_body(nacc, ncnt, eacc, ecnt, u, w1, b1, w2, b2, out):
    nsum = nacc[0] + nacc[1]
    esum = eacc[0] + eacc[1]
    ncount = jnp.sum(ncnt[...], axis=(0, 1)).reshape(B, 1)
    ecount = jnp.sum(ecnt[...], axis=(0, 1)).reshape(B, 1)
    nmean = nsum / jnp.maximum(ncount, 1.0)
    emean = esum / jnp.maximum(ecount, 1.0)
    U = w1.shape[0] - 2 * D
    h = (jnp.dot(u[...], w1[0:U], preferred_element_type=jnp.float32)
         + jnp.dot(nmean, w1[U:U + D], preferred_element_type=jnp.float32)
         + jnp.dot(emean, w1[U + D:], preferred_element_type=jnp.float32)
         + b1[...])
    h = jnp.maximum(h, 0.0)
    out[...] = jnp.dot(h, w2[...], preferred_element_type=jnp.float32) + b2[...]


def kernel(x, edge_index, edge_attr, u, batch, W1, b1, W2, b2):
    ei = edge_index if edge_index.dtype == jnp.int32 else edge_index.astype(jnp.int32)
    ei = ei.reshape(-1)
    batch_i = batch if batch.dtype == jnp.int32 else batch.astype(jnp.int32)
    nacc, ncnt, eacc, ecnt = _sc_segment_sums(ei, x, batch_i, edge_attr)

    out = pl.pallas_call(
        _mlp_body,
        out_shape=jax.ShapeDtypeStruct((B, D), jnp.float32),
    )(nacc, ncnt, eacc, ecnt, u, W1,
      b1.reshape(1, D), W2, b2.reshape(1, D))
    return out


# speedup vs baseline: 24.1272x; 1.1202x over previous
import functools

import jax
import jax.numpy as jnp
from jax import lax
from jax.experimental import pallas as pl
from jax.experimental.pallas import tpu as pltpu
from jax.experimental.pallas import tpu_sc as plsc

B = 64
D = 128
N = 10000
E = 320000
NC = 2
NS = 16
NW = NC * NS

SUB = 80
NSUB = 5
CHUNK = SUB * NSUB
E_PER_W = E // NW
N_ECHUNKS = E_PER_W // CHUNK
N_NBLOCKS = N // SUB


def _histo_update(cnt_ref, seg_vec):
    r, m = plsc.scan_count(seg_vec)
    plsc.addupdate_scatter(cnt_ref, [seg_vec], r.astype(jnp.float32), mask=m)


def _sc_body(ei_hbm, x_hbm, batch_hbm, eattr_hbm,
             nacc_out, ncnt_out, eacc_out, ecnt_out,
             batch_v, ei_v0, ei_v1, attr_v0, attr_v1, xstage_v,
             sg00, sg01, sg02, sg03, sg04,
             sg10, sg11, sg12, sg13, sg14,
             ncnt_v, ecnt_v, nacc_s, eacc_s,
             gsem0, gsem1, ssem0, ssem1, eisem0, eisem1):
    c = lax.axis_index("c")
    s = lax.axis_index("s")
    w = s * NC + c

    ei_v = (ei_v0, ei_v1)
    attr_v = (attr_v0, attr_v1)
    seg_v = ((sg00, sg01, sg02, sg03, sg04),
             (sg10, sg11, sg12, sg13, sg14))
    nseg_v = (sg00, sg01, sg02, sg03)
    gsem = (gsem0, gsem1)
    ssem = (ssem0, ssem1)
    eisem = (eisem0, eisem1)
    ebase = w * E_PER_W
    RPT = B // NS

    def start_gather(k, b):
        cb = ebase + k * CHUNK
        pltpu.async_copy(ei_hbm.at[pl.ds(E + cb, CHUNK)], ei_v[b], eisem[b])
        pltpu.async_copy(eattr_hbm.at[pl.ds(cb, CHUNK)], attr_v[b], gsem[b])

    def wait_gather_ei(k, b):
        cb = ebase + k * CHUNK
        pltpu.make_async_copy(ei_hbm.at[pl.ds(E + cb, CHUNK)], ei_v[b],
                              eisem[b]).wait()

    def wait_gather_attr(k, b):
        cb = ebase + k * CHUNK
        pltpu.make_async_copy(eattr_hbm.at[pl.ds(cb, CHUNK)], attr_v[b],
                              gsem[b]).wait()

    start_gather(0, 0)

    NB_IT = (N_NBLOCKS + NW - 1) // NW

    for it in range(NB_IT):
        @pl.when(w + it * NW < N_NBLOCKS)
        def _(it=it):
            base = (w + it * NW) * SUB
            pltpu.async_copy(batch_hbm.at[pl.ds(base, SUB)], nseg_v[it],
                             gsem1)
            pltpu.async_copy(x_hbm.at[pl.ds(base, SUB)],
                             attr_v1.at[pl.ds(it * SUB, SUB)], gsem1)

    zero16 = jnp.zeros((16,), jnp.float32)
    for t in range(B // 16):
        ncnt_v[pl.ds(t * 16, 16)] = zero16
        ecnt_v[pl.ds(t * 16, 16)] = zero16
    for i in range(RPT):
        for j in range(D // 16):
            xstage_v[i, pl.ds(j * 16, 16)] = zero16

    pltpu.sync_copy(batch_hbm, batch_v)

    pltpu.sync_copy(xstage_v.at[pl.ds(0, RPT)], nacc_s.at[pl.ds(s * RPT, RPT)])
    pltpu.sync_copy(xstage_v.at[pl.ds(0, RPT)], eacc_s.at[pl.ds(s * RPT, RPT)])

    plsc.subcore_barrier()

    for it in range(NB_IT):
        @pl.when(w + it * NW < N_NBLOCKS)
        def _(it=it):
            base = (w + it * NW) * SUB
            pltpu.make_async_copy(batch_hbm.at[pl.ds(base, SUB)], nseg_v[it],
                                  gsem1).wait()
            pltpu.make_async_copy(x_hbm.at[pl.ds(base, SUB)],
                                  attr_v1.at[pl.ds(it * SUB, SUB)],
                                  gsem1).wait()
            pltpu.async_copy(attr_v1.at[pl.ds(it * SUB, SUB)],
                             nacc_s.at[nseg_v[it]], ssem0, add=True)
            for t in range(SUB // 16):
                _histo_update(ncnt_v, nseg_v[it][pl.ds(t * 16, 16)])

    for it in range(NB_IT):
        @pl.when(w + it * NW < N_NBLOCKS)
        def _(it=it):
            pltpu.make_async_copy(attr_v1.at[pl.ds(it * SUB, SUB)],
                                  nacc_s.at[nseg_v[it]], ssem0).wait()

    start_gather(1, 1)


    def compute_segs(b):
        for j in range(NSUB):
            for t in range(SUB // 16):
                idx = ei_v[b][pl.ds(j * SUB + t * 16, 16)]
                seg = plsc.load_gather(batch_v, [idx])
                seg_v[b][j][pl.ds(t * 16, 16)] = seg
                _histo_update(ecnt_v, seg)

    def issue_scatter(b):
        for j in range(NSUB):
            pltpu.async_copy(attr_v[b].at[pl.ds(j * SUB, SUB)],
                             eacc_s.at[seg_v[b][j]], ssem[b], add=True)

    def drain_scatter(b):
        for j in range(NSUB):
            pltpu.make_async_copy(attr_v[b].at[pl.ds(j * SUB, SUB)],
                                  eacc_s.at[seg_v[b][j]], ssem[b]).wait()

    wait_gather_ei(0, 0)
    compute_segs(0)
    wait_gather_attr(0, 0)
    issue_scatter(0)

    NPAIR = (N_ECHUNKS - 1) // 2

    def epair(k2, _):
        ka = 2 * k2 + 1
        wait_gather_ei(ka, 1)
        compute_segs(1)
        wait_gather_attr(ka, 1)
        issue_scatter(1)
        drain_scatter(0)
        start_gather(ka + 1, 0)

        kb = ka + 1
        wait_gather_ei(kb, 0)
        compute_segs(0)
        wait_gather_attr(kb, 0)
        issue_scatter(0)
        drain_scatter(1)

        @pl.when(k2 < NPAIR - 1)
        def _():
            start_gather(kb + 1, 1)
        return 0

    lax.fori_loop(0, NPAIR, epair, 0)
    drain_scatter(0)

    pltpu.sync_copy(ncnt_v, ncnt_out.at[c, s])
    pltpu.sync_copy(ecnt_v, ecnt_out.at[c, s])

    plsc.subcore_barrier()

    pltpu.sync_copy(nacc_s.at[pl.ds(s * RPT, RPT)],
                    nacc_out.at[c, pl.ds(s * RPT, RPT)])
    pltpu.sync_copy(eacc_s.at[pl.ds(s * RPT, RPT)],
                    eacc_out.at[c, pl.ds(s * RPT, RPT)])


_sc_segment_sums = functools.partial(
    pl.kernel,
    out_type=(
        jax.ShapeDtypeStruct((NC, B, D), jnp.float32),
        jax.ShapeDtypeStruct((NC, NS, B), jnp.float32),
        jax.ShapeDtypeStruct((NC, B, D), jnp.float32),
        jax.ShapeDtypeStruct((NC, NS, B), jnp.float32),
    ),
    mesh=plsc.VectorSubcoreMesh(core_axis_name="c", subcore_axis_name="s"),
    compiler_params=pltpu.CompilerParams(needs_layout_passes=False),
    scratch_types=[
        pltpu.VMEM((N,), jnp.int32),
        pltpu.VMEM((CHUNK,), jnp.int32),
        pltpu.VMEM((CHUNK,), jnp.int32),
        pltpu.VMEM((CHUNK, D), jnp.float32),
        pltpu.VMEM((CHUNK, D), jnp.float32),
        pltpu.VMEM((B // NS, D), jnp.float32),
    ] + [pltpu.VMEM((SUB,), jnp.int32)] * (2 * NSUB)
    + [
        pltpu.VMEM((B,), jnp.float32),
        pltpu.VMEM((B,), jnp.float32),
        pltpu.VMEM_SHARED((B, D), jnp.float32),
        pltpu.VMEM_SHARED((B, D), jnp.float32),
        pltpu.SemaphoreType.DMA,
        pltpu.SemaphoreType.DMA,
        pltpu.SemaphoreType.DMA,
        pltpu.SemaphoreType.DMA,
        pltpu.SemaphoreType.DMA,
        pltpu.SemaphoreType.DMA,
    ],
)(_sc_body)


def _mlp_body(nacc, ncnt, eacc, ecnt, u, w1, b1, w2, b2, out):
    nsum = nacc[0] + nacc[1]
    esum = eacc[0] + eacc[1]
    ncount = jnp.sum(ncnt[...], axis=(0, 1)).reshape(B, 1)
    ecount = jnp.sum(ecnt[...], axis=(0, 1)).reshape(B, 1)
    nmean = nsum / jnp.maximum(ncount, 1.0)
    emean = esum / jnp.maximum(ecount, 1.0)
    U = w1.shape[0] - 2 * D
    h = (jnp.dot(u[...], w1[0:U], preferred_element_type=jnp.float32)
         + jnp.dot(nmean, w1[U:U + D], preferred_element_type=jnp.float32)
         + jnp.dot(emean, w1[U + D:], preferred_element_type=jnp.float32)
         + b1[...])
    h = jnp.maximum(h, 0.0)
    out[...] = jnp.dot(h, w2[...], preferred_element_type=jnp.float32) + b2[...]


def kernel(x, edge_index, edge_attr, u, batch, W1, b1, W2, b2):
    ei = edge_index if edge_index.dtype == jnp.int32 else edge_index.astype(jnp.int32)
    ei = ei.reshape(-1)
    batch_i = batch if batch.dtype == jnp.int32 else batch.astype(jnp.int32)
    nacc, ncnt, eacc, ecnt = _sc_segment_sums(ei, x, batch_i, edge_attr)

    out = pl.pallas_call(
        _mlp_body,
        out_shape=jax.ShapeDtypeStruct((B, D), jnp.float32),
    )(nacc, ncnt, eacc, ecnt, u, W1,
      b1.reshape(1, D), W2, b2.reshape(1, D))
    return out
